# baseline (speedup 1.0000x reference)
import dataclasses
import functools

import jax
import jax.numpy as jnp
from jax import lax
from jax.experimental import pallas as pl
from jax.experimental.pallas import tpu as pltpu
from jax.experimental.pallas import tpu_sc as plsc

B = 16384
L = 50
LP = 52
GR = 2 * LP
D = 32
NW = 32
IPW = B // NW
TOK_GROUPS = IPW // 2
ART_GROUPS = IPW // 128
NBUF = 4
TEXT_V = 10000
STAGE = TEXT_V // 16


def _sc_item_model(art_idx, tok_idx, artist_table, text_table):
    mesh = plsc.VectorSubcoreMesh(core_axis_name="c", subcore_axis_name="s")

    cp = pltpu.CompilerParams()
    if "needs_layout_passes" in pltpu.CompilerParams.__dataclass_fields__:
        cp = dataclasses.replace(cp, needs_layout_passes=False)
    if "use_tc_tiling_on_sc" in pltpu.CompilerParams.__dataclass_fields__:
        cp = dataclasses.replace(cp, use_tc_tiling_on_sc=False)

    @functools.partial(
        pl.kernel,
        compiler_params=cp,
        out_type=jax.ShapeDtypeStruct((B, 2 * D), jnp.float32),
        mesh=mesh,
        scratch_types=[
            pltpu.VMEM((TOK_GROUPS, GR), jnp.int32),
            pltpu.VMEM((ART_GROUPS, 128), jnp.int32),
            pltpu.VMEM((NBUF, GR, D), jnp.bfloat16),
            pltpu.VMEM((IPW, D), jnp.float32),
            pltpu.VMEM((IPW, 2 * D), jnp.float32),
            pltpu.VMEM((STAGE, D), jnp.bfloat16),
            pltpu.VMEM_SHARED((TEXT_V, D), jnp.bfloat16),
            pltpu.SemaphoreType.DMA((NBUF,)),
            pltpu.SemaphoreType.DMA,
        ],
    )
    def kern(art_hbm, tok_hbm, atab_hbm, ttab_hbm, out_hbm,
             tok_v, art_v, gring, abuf, obuf, stage_v, ttab_sh, gsem, asem):
        c = lax.axis_index("c")
        s = lax.axis_index("s")
        w = s * 2 + c

        pltpu.sync_copy(tok_hbm.at[pl.ds(w * TOK_GROUPS, TOK_GROUPS)], tok_v)
        pltpu.sync_copy(art_hbm.at[pl.ds(w * ART_GROUPS, ART_GROUPS)], art_v)

        for j in range(ART_GROUPS):
            pltpu.make_async_copy(atab_hbm.at[art_v.at[j]],
                                  abuf.at[pl.ds(j * 128, 128)], asem).start()

        off = s * STAGE
        pltpu.sync_copy(ttab_hbm.at[pl.ds(off, STAGE)], stage_v)
        pltpu.sync_copy(stage_v, ttab_sh.at[pl.ds(off, STAGE)])
        plsc.subcore_barrier()

        def fire(g, b):
            is_hbm = g % 11 == 10

            @pl.when(is_hbm)
            def _():
                pltpu.make_async_copy(ttab_hbm.at[tok_v.at[g]], gring.at[b],
                                      gsem.at[b]).start()

            @pl.when(jnp.logical_not(is_hbm))
            def _():
                pltpu.make_async_copy(ttab_sh.at[tok_v.at[g]], gring.at[b],
                                      gsem.at[b]).start()

        def drain(b):
            pltpu.make_async_copy(ttab_sh.at[tok_v.at[0]], gring.at[b],
                                  gsem.at[b]).wait()

        for b in range(NBUF - 1):
            fire(b, b)
        for j in range(ART_GROUPS):
            pltpu.make_async_copy(atab_hbm.at[art_v.at[j]],
                                  abuf.at[pl.ds(j * 128, 128)], asem).wait()

        iota = lax.iota(jnp.int32, 16)

        def process(g, gbuf):
            for it in range(2):
                base = it * LP

                def body(i, accs):
                    a0, a1, a2, a3, b0, b1, b2, b3 = accs
                    r = base + 4 * i
                    lo0, hi0 = plsc.unpack(gbuf[r + 0, :],
                                           format=plsc.PackFormat.INTERLEAVED)
                    lo1, hi1 = plsc.unpack(gbuf[r + 1, :],
                                           format=plsc.PackFormat.INTERLEAVED)
                    lo2, hi2 = plsc.unpack(gbuf[r + 2, :],
                                           format=plsc.PackFormat.INTERLEAVED)
                    lo3, hi3 = plsc.unpack(gbuf[r + 3, :],
                                           format=plsc.PackFormat.INTERLEAVED)
                    a0 = a0 + lo0
                    a1 = a1 + lo1
                    a2 = a2 + lo2
                    a3 = a3 + lo3
                    b0 = b0 + hi0
                    b1 = b1 + hi1
                    b2 = b2 + hi2
                    b3 = b3 + hi3
                    return a0, a1, a2, a3, b0, b1, b2, b3

                z = jnp.zeros((16,), jnp.float32)
                a0, a1, a2, a3, b0, b1, b2, b3 = lax.fori_loop(
                    0, LP // 4, body, (z, z, z, z, z, z, z, z))
                sum_lo = (a0 + a1) + (a2 + a3)
                sum_hi = (b0 + b1) + (b2 + b3)

                if it == 0:
                    spans = ((0, None), (16, None), (32, None), (48, iota < 4))
                else:
                    spans = ((48, iota >= 4), (64, None), (80, None),
                             (88, iota >= 8))
                cvec = jnp.zeros((16,), jnp.int32)
                for k, cond in spans:
                    t = tok_v[g, pl.ds(k, 16)]
                    nz = (t != 0).astype(jnp.int32)
                    if cond is not None:
                        nz = jnp.where(cond, nz, 0)
                    cvec = cvec + nz
                cnt = jnp.maximum(jnp.sum(cvec).astype(jnp.float32), 1.0)

                item = g * 2 + it
                obuf[item, pl.ds(0, 16)] = abuf[item, pl.ds(0, 16)]
                obuf[item, pl.ds(16, 16)] = abuf[item, pl.ds(16, 16)]
                obuf[item, pl.ds(D, 16)] = sum_lo / cnt
                obuf[item, pl.ds(D + 16, 16)] = sum_hi / cnt

        @pl.loop(0, TOK_GROUPS, step=NBUF)
        def _(g):
            for b in range(NBUF):
                gg = g + b
                nxt = gg + (NBUF - 1)

                @pl.when(nxt < TOK_GROUPS)
                def _():
                    fire(nxt, (b + NBUF - 1) % NBUF)

                drain(b)
                process(gg, gring.at[b])

        pltpu.sync_copy(obuf, out_hbm.at[pl.ds(w * IPW, IPW)])

    return kern(art_idx, tok_idx, artist_table, text_table)


def kernel(artist_ids, genre_tokens, artist_table, text_table):
    tok = jnp.pad(genre_tokens, ((0, 0), (0, LP - L)))
    tok_idx = tok.reshape(B // 2, GR)
    art_idx = artist_ids.reshape(B // 128, 128)
    text_z = text_table.at[0].set(0.0)
    perm = jnp.arange(D).reshape(2, D // 2).T.reshape(-1)
    text_bf = text_z[:, perm].astype(jnp.bfloat16)
    return _sc_item_model(art_idx, tok_idx, artist_table, text_bf)


# speedup vs baseline: 1.0397x; 1.0397x over previous
import dataclasses
import functools

import jax
import jax.numpy as jnp
from jax import lax
from jax.experimental import pallas as pl
from jax.experimental.pallas import tpu as pltpu
from jax.experimental.pallas import tpu_sc as plsc

B = 16384
L = 50
LP = 52
GR = 2 * LP
D = 32
NW = 32
IPW = B // NW
TOK_GROUPS = IPW // 2
ART_GROUPS = IPW // 128
NBUF = 4
TEXT_V = 10000
STAGE = TEXT_V // 16


def _sc_item_model(art_idx, tok_idx, artist_table, text_table):
    mesh = plsc.VectorSubcoreMesh(core_axis_name="c", subcore_axis_name="s")

    cp = pltpu.CompilerParams()
    if "needs_layout_passes" in pltpu.CompilerParams.__dataclass_fields__:
        cp = dataclasses.replace(cp, needs_layout_passes=False)
    if "use_tc_tiling_on_sc" in pltpu.CompilerParams.__dataclass_fields__:
        cp = dataclasses.replace(cp, use_tc_tiling_on_sc=False)

    @functools.partial(
        pl.kernel,
        compiler_params=cp,
        out_type=jax.ShapeDtypeStruct((B, 2 * D), jnp.float32),
        mesh=mesh,
        scratch_types=[
            pltpu.VMEM((TOK_GROUPS, GR), jnp.int32),
            pltpu.VMEM((ART_GROUPS, 128), jnp.int32),
            pltpu.VMEM((NBUF, GR, D), jnp.bfloat16),
            pltpu.VMEM((IPW, D), jnp.float32),
            pltpu.VMEM((IPW, 2 * D), jnp.float32),
            pltpu.VMEM((STAGE, D), jnp.bfloat16),
            pltpu.VMEM_SHARED((TEXT_V, D), jnp.bfloat16),
            pltpu.SemaphoreType.DMA((NBUF,)),
            pltpu.SemaphoreType.DMA,
        ],
    )
    def kern(art_hbm, tok_hbm, atab_hbm, ttab_hbm, out_hbm,
             tok_v, art_v, gring, abuf, obuf, stage_v, ttab_sh, gsem, asem):
        c = lax.axis_index("c")
        s = lax.axis_index("s")
        w = s * 2 + c

        pltpu.sync_copy(tok_hbm.at[pl.ds(w * TOK_GROUPS, TOK_GROUPS)], tok_v)
        pltpu.sync_copy(art_hbm.at[pl.ds(w * ART_GROUPS, ART_GROUPS)], art_v)

        for j in range(ART_GROUPS):
            pltpu.make_async_copy(atab_hbm.at[art_v.at[j]],
                                  abuf.at[pl.ds(j * 128, 128)], asem).start()

        off = s * STAGE
        pltpu.sync_copy(ttab_hbm.at[pl.ds(off, STAGE)], stage_v)
        pltpu.sync_copy(stage_v, ttab_sh.at[pl.ds(off, STAGE)])
        plsc.subcore_barrier()

        def fire(g, b):
            pltpu.make_async_copy(ttab_sh.at[tok_v.at[g]], gring.at[b],
                                  gsem.at[b]).start()

        def drain(b):
            pltpu.make_async_copy(ttab_sh.at[tok_v.at[0]], gring.at[b],
                                  gsem.at[b]).wait()

        for b in range(NBUF - 1):
            fire(b, b)
        for j in range(ART_GROUPS):
            pltpu.make_async_copy(atab_hbm.at[art_v.at[j]],
                                  abuf.at[pl.ds(j * 128, 128)], asem).wait()

        iota = lax.iota(jnp.int32, 16)

        def process(g, gbuf):
            for it in range(2):
                base = it * LP

                def body(i, accs):
                    a0, a1, a2, a3, b0, b1, b2, b3 = accs
                    r = base + 4 * i
                    lo0, hi0 = plsc.unpack(gbuf[r + 0, :],
                                           format=plsc.PackFormat.INTERLEAVED)
                    lo1, hi1 = plsc.unpack(gbuf[r + 1, :],
                                           format=plsc.PackFormat.INTERLEAVED)
                    lo2, hi2 = plsc.unpack(gbuf[r + 2, :],
                                           format=plsc.PackFormat.INTERLEAVED)
                    lo3, hi3 = plsc.unpack(gbuf[r + 3, :],
                                           format=plsc.PackFormat.INTERLEAVED)
                    a0 = a0 + lo0
                    a1 = a1 + lo1
                    a2 = a2 + lo2
                    a3 = a3 + lo3
                    b0 = b0 + hi0
                    b1 = b1 + hi1
                    b2 = b2 + hi2
                    b3 = b3 + hi3
                    return a0, a1, a2, a3, b0, b1, b2, b3

                z = jnp.zeros((16,), jnp.float32)
                a0, a1, a2, a3, b0, b1, b2, b3 = lax.fori_loop(
                    0, LP // 4, body, (z, z, z, z, z, z, z, z))
                sum_lo = (a0 + a1) + (a2 + a3)
                sum_hi = (b0 + b1) + (b2 + b3)

                if it == 0:
                    spans = ((0, None), (16, None), (32, None), (48, iota < 4))
                else:
                    spans = ((48, iota >= 4), (64, None), (80, None),
                             (88, iota >= 8))
                cvec = jnp.zeros((16,), jnp.int32)
                for k, cond in spans:
                    t = tok_v[g, pl.ds(k, 16)]
                    nz = (t != 0).astype(jnp.int32)
                    if cond is not None:
                        nz = jnp.where(cond, nz, 0)
                    cvec = cvec + nz
                cnt = jnp.maximum(jnp.sum(cvec).astype(jnp.float32), 1.0)

                item = g * 2 + it
                obuf[item, pl.ds(0, 16)] = abuf[item, pl.ds(0, 16)]
                obuf[item, pl.ds(16, 16)] = abuf[item, pl.ds(16, 16)]
                obuf[item, pl.ds(D, 16)] = sum_lo / cnt
                obuf[item, pl.ds(D + 16, 16)] = sum_hi / cnt

        @pl.loop(0, TOK_GROUPS, step=NBUF)
        def _(g):
            for b in range(NBUF):
                gg = g + b
                nxt = gg + (NBUF - 1)

                @pl.when(nxt < TOK_GROUPS)
                def _():
                    fire(nxt, (b + NBUF - 1) % NBUF)

                drain(b)
                process(gg, gring.at[b])

        pltpu.sync_copy(obuf, out_hbm.at[pl.ds(w * IPW, IPW)])

    return kern(art_idx, tok_idx, artist_table, text_table)


def kernel(artist_ids, genre_tokens, artist_table, text_table):
    tok = jnp.pad(genre_tokens, ((0, 0), (0, LP - L)))
    tok_idx = tok.reshape(B // 2, GR)
    art_idx = artist_ids.reshape(B // 128, 128)
    text_z = text_table.at[0].set(0.0)
    perm = jnp.arange(D).reshape(2, D // 2).T.reshape(-1)
    text_bf = text_z[:, perm].astype(jnp.bfloat16)
    return _sc_item_model(art_idx, tok_idx, artist_table, text_bf)
